# baseline (speedup 1.0000x reference)
import jax
import jax.numpy as jnp
from jax import lax
from jax.experimental import pallas as pl
from jax.experimental.pallas import tpu as pltpu
from jax.experimental.pallas import tpu_sc as plsc

B = 16384
D = 128
LANES = 16
NC = 2
NS = 16
NW = NC * NS
BPW = B // NW
CH = 128
NCHUNK = BPW // CH


def _sc_body(user_h, item_h, theta_h, a_h, b_h, out_h,
             uidx, iidx, tb0, ab0, bb0, tb1, ab1, bb1, tb2, ab2, bb2,
             obuf, accb, sem0, sem1, sem2):
    wid = lax.axis_index("s") * NC + lax.axis_index("c")
    h_u = pltpu.async_copy(user_h.at[wid], uidx, sem0)
    h_i = pltpu.async_copy(item_h.at[wid], iidx, sem1)
    h_u.wait()
    h_i.wait()

    bufs = ((tb0, ab0, bb0, sem0), (tb1, ab1, bb1, sem1),
            (tb2, ab2, bb2, sem2))

    def start(c):
        tb, ab, bb, sem = bufs[c % 3]
        return (pltpu.async_copy(theta_h.at[uidx.at[c]], tb, sem),
                pltpu.async_copy(a_h.at[iidx.at[c]], ab, sem),
                pltpu.async_copy(b_h.at[iidx.at[c]], bb, sem))

    lane = lax.iota(jnp.int32, LANES)
    handles = [start(0), start(1)]

    for c in range(NCHUNK):
        if c + 2 < NCHUNK:
            handles.append(start(c + 2))
        for h in handles[c]:
            h.wait()
        tb, ab, bb, _ = bufs[c % 3]

        @plsc.parallel_loop(0, CH)
        def persample(s, tb=tb, ab=ab):
            num, den = None, None
            for q in range(D // LANES // 2):
                r0, r1 = 2 * q, 2 * q + 1
                av0 = ab[s, pl.ds(r0 * LANES, LANES)]
                av1 = ab[s, pl.ds(r1 * LANES, LANES)]
                th0 = tb[s, pl.ds(r0 * LANES, LANES)]
                th1 = tb[s, pl.ds(r1 * LANES, LANES)]
                p = plsc.pack(av0, av1, format=plsc.PackFormat.INTERLEAVED)
                e = jnp.exp(-p)
                u0, u1 = plsc.unpack(e, format=plsc.PackFormat.INTERLEAVED)
                d0 = 1.0 + u0
                d1 = 1.0 + u1
                n_q = th0 * d1 + th1 * d0
                d_q = d0 * d1
                if num is None:
                    num, den = n_q, d_q
                else:
                    num = num * d_q + n_q * den
                    den = den * d_q
            accb[s, pl.ds(0, LANES)] = num / den

        del persample

        @plsc.parallel_loop(0, CH // LANES)
        def group(g, bb=bb, c=c):
            rows = g * LANES + lane
            cols = [plsc.load_gather(accb, [rows, jnp.full((LANES,), k, jnp.int32)])
                    for k in range(LANES)]
            while len(cols) > 1:
                cols = [cols[i] + cols[i + 1] for i in range(0, len(cols), 2)]
            logit = cols[0] - bb[pl.ds(g * LANES, LANES)]
            obuf[c, pl.ds(g * LANES, LANES)] = 1.0 / (1.0 + jnp.exp(-logit))

        del group

    pltpu.sync_copy(obuf, out_h.at[pl.ds(wid * NCHUNK, NCHUNK)])


def kernel(user, item, theta_w, a_w, b_w):
    user3 = user.reshape(NW, NCHUNK, CH)
    item3 = item.reshape(NW, NCHUNK, CH)
    b_w = lax.reshape(b_w, (b_w.shape[0],), dimensions=(1, 0))
    mesh = plsc.VectorSubcoreMesh(core_axis_name="c", subcore_axis_name="s")
    run = pl.kernel(
        _sc_body,
        mesh=mesh,
        out_type=jax.ShapeDtypeStruct((NW * NCHUNK, CH), jnp.float32),
        scratch_types=[
            pltpu.VMEM((NCHUNK, CH), jnp.int32),
            pltpu.VMEM((NCHUNK, CH), jnp.int32),
            pltpu.VMEM((CH, D), jnp.float32),
            pltpu.VMEM((CH, D), jnp.float32),
            pltpu.VMEM((CH,), jnp.float32),
            pltpu.VMEM((CH, D), jnp.float32),
            pltpu.VMEM((CH, D), jnp.float32),
            pltpu.VMEM((CH,), jnp.float32),
            pltpu.VMEM((CH, D), jnp.float32),
            pltpu.VMEM((CH, D), jnp.float32),
            pltpu.VMEM((CH,), jnp.float32),
            pltpu.VMEM((NCHUNK, CH), jnp.float32),
            pltpu.VMEM((CH, LANES + 1), jnp.float32),
            pltpu.SemaphoreType.DMA,
            pltpu.SemaphoreType.DMA,
            pltpu.SemaphoreType.DMA,
        ],
        compiler_params=pltpu.CompilerParams(
            needs_layout_passes=False,
            skip_device_barrier=True,
            disable_bounds_checks=True,
            disable_semaphore_checks=True,
        ),
    )
    out = run(user3, item3, theta_w, a_w, b_w)
    return out.reshape(B)


# speedup vs baseline: 1.0562x; 1.0562x over previous
import jax
import jax.numpy as jnp
from jax import lax
from jax.experimental import pallas as pl
from jax.experimental.pallas import tpu as pltpu
from jax.experimental.pallas import tpu_sc as plsc

B = 16384
D = 128
LANES = 16
NC = 2
NS = 16
NW = NC * NS
BPW = B // NW
CH = 128
NCHUNK = BPW // CH


def _sc_body(user_h, item_h, theta_h, a_h, b_h, out_h,
             uidx, iidx, tb0, ab0, bb0, tb1, ab1, bb1, obuf, accb,
             sem0, sem1):
    wid = lax.axis_index("s") * NC + lax.axis_index("c")
    h_u = pltpu.async_copy(user_h.at[wid], uidx, sem0)
    h_i = pltpu.async_copy(item_h.at[wid], iidx, sem1)
    h_u.wait()
    h_i.wait()

    bufs = ((tb0, ab0, bb0, sem0), (tb1, ab1, bb1, sem1))

    def start(c):
        tb, ab, bb, sem = bufs[c % 2]
        return (pltpu.async_copy(theta_h.at[uidx.at[c]], tb, sem),
                pltpu.async_copy(a_h.at[iidx.at[c]], ab, sem),
                pltpu.async_copy(b_h.at[iidx.at[c]], bb, sem))

    lane = lax.iota(jnp.int32, LANES)
    handles = [start(0)]

    for c in range(NCHUNK):
        if c + 1 < NCHUNK:
            handles.append(start(c + 1))
        for h in handles[c]:
            h.wait()
        tb, ab, bb, _ = bufs[c % 2]

        @plsc.parallel_loop(0, CH)
        def persample(s, tb=tb, ab=ab):
            num, den = None, None
            for q in range(D // LANES // 2):
                r0, r1 = 2 * q, 2 * q + 1
                av0 = ab[s, pl.ds(r0 * LANES, LANES)]
                av1 = ab[s, pl.ds(r1 * LANES, LANES)]
                th0 = tb[s, pl.ds(r0 * LANES, LANES)]
                th1 = tb[s, pl.ds(r1 * LANES, LANES)]
                p = plsc.pack(av0, av1, format=plsc.PackFormat.INTERLEAVED)
                e = jnp.exp(-p)
                u0, u1 = plsc.unpack(e, format=plsc.PackFormat.INTERLEAVED)
                d0 = 1.0 + u0
                d1 = 1.0 + u1
                n_q = th0 * d1 + th1 * d0
                d_q = d0 * d1
                if num is None:
                    num, den = n_q, d_q
                else:
                    num = num * d_q + n_q * den
                    den = den * d_q
            accb[s, pl.ds(0, LANES)] = num / den

        del persample

        @plsc.parallel_loop(0, CH // LANES)
        def group(g, bb=bb, c=c):
            rows = g * LANES + lane
            cols = [plsc.load_gather(accb, [rows, jnp.full((LANES,), k, jnp.int32)])
                    for k in range(LANES)]
            while len(cols) > 1:
                cols = [cols[i] + cols[i + 1] for i in range(0, len(cols), 2)]
            logit = cols[0] - bb[pl.ds(g * LANES, LANES)]
            obuf[c, pl.ds(g * LANES, LANES)] = 1.0 / (1.0 + jnp.exp(-logit))

        del group

    pltpu.sync_copy(obuf, out_h.at[pl.ds(wid * NCHUNK, NCHUNK)])


def kernel(user, item, theta_w, a_w, b_w):
    user3 = user.reshape(NW, NCHUNK, CH)
    item3 = item.reshape(NW, NCHUNK, CH)
    b_w = lax.reshape(b_w, (b_w.shape[0],), dimensions=(1, 0))
    mesh = plsc.VectorSubcoreMesh(core_axis_name="c", subcore_axis_name="s")
    run = pl.kernel(
        _sc_body,
        mesh=mesh,
        out_type=jax.ShapeDtypeStruct((NW * NCHUNK, CH), jnp.float32),
        scratch_types=[
            pltpu.VMEM((NCHUNK, CH), jnp.int32),
            pltpu.VMEM((NCHUNK, CH), jnp.int32),
            pltpu.VMEM((CH, D), jnp.float32),
            pltpu.VMEM((CH, D), jnp.float32),
            pltpu.VMEM((CH,), jnp.float32),
            pltpu.VMEM((CH, D), jnp.float32),
            pltpu.VMEM((CH, D), jnp.float32),
            pltpu.VMEM((CH,), jnp.float32),
            pltpu.VMEM((NCHUNK, CH), jnp.float32),
            pltpu.VMEM((CH, LANES + 1), jnp.float32),
            pltpu.SemaphoreType.DMA,
            pltpu.SemaphoreType.DMA,
        ],
        compiler_params=pltpu.CompilerParams(
            needs_layout_passes=False,
            skip_device_barrier=True,
            disable_bounds_checks=True,
            disable_semaphore_checks=True,
        ),
    )
    out = run(user3, item3, theta_w, a_w, b_w)
    return out.reshape(B)
